# Initial kernel scaffold; baseline (speedup 1.0000x reference)
import jax
import jax.numpy as jnp
from jax.experimental import pallas as pl


def kernel(u_emb, i_emb, edge_index):
    raise NotImplementedError("write your pallas kernel here")


# speedup vs baseline: 8.4020x; 8.4020x over previous
import functools

import jax
import jax.numpy as jnp
from jax import lax
from jax.experimental import pallas as pl
from jax.experimental.pallas import tpu as pltpu
from jax.experimental.pallas import tpu_sc as plsc

N_NODES = 50000
DIM = 64
N_EDGES = 800000
HALF = 25000
NP = 50176
RB = 6272
CH = 128
NCH = 391
EP = 16 * NCH * CH
R_ACC = 26624
STRIPE = R_ACC // 16
TAIL = HALF - 15 * STRIPE
DEG_T = 65536
DSTRIPE = DEG_T // 16

_mesh = plsc.VectorSubcoreMesh(core_axis_name="c", subcore_axis_name="s")
_cp = pltpu.CompilerParams(use_tc_tiling_on_sc=False)


@functools.partial(
    pl.kernel,
    out_type=jax.ShapeDtypeStruct((2, DEG_T), jnp.float32),
    mesh=_mesh,
    scratch_types=[
        pltpu.VMEM((DSTRIPE,), jnp.float32),
        pltpu.VMEM((CH,), jnp.int32),
        pltpu.VMEM((1, CH), jnp.int32),
        pltpu.VMEM((CH,), jnp.float32),
        pltpu.VMEM_SHARED((DEG_T,), jnp.float32),
    ],
)
def _degree_kernel(e_hbm, deg_hbm, zbuf, ibuf, xbuf, ones, acc):
    c = lax.axis_index("c")
    s = lax.axis_index("s")

    @pl.loop(0, DSTRIPE // 16)
    def _(i):
        zbuf[pl.ds(i * 16, 16)] = jnp.zeros((16,), jnp.float32)

    for j in range(CH // 16):
        ones[pl.ds(j * 16, 16)] = jnp.ones((16,), jnp.float32)

    pltpu.sync_copy(zbuf, acc.at[pl.ds(s * DSTRIPE, DSTRIPE)])
    plsc.subcore_barrier()

    lane = lax.iota(jnp.int32, 16)

    @pl.loop(0, NCH)
    def _(i):
        base = s * (NCH * CH) + i * CH
        pltpu.sync_copy(e_hbm.at[c, pl.ds(base, CH)], ibuf)
        for j in range(CH // 16):
            v = ibuf[pl.ds(j * 16, 16)]
            ok = (v >= 0) & (v < N_NODES)
            trash = N_NODES + lane + j * 16
            xbuf[0, pl.ds(j * 16, 16)] = jnp.where(ok, v, trash)
        pltpu.sync_copy(ones, acc.at[xbuf.at[0]], add=True)

    plsc.subcore_barrier()
    pltpu.sync_copy(acc.at[pl.ds(s * DSTRIPE, DSTRIPE)], zbuf)
    pltpu.sync_copy(zbuf, deg_hbm.at[c, pl.ds(s * DSTRIPE, DSTRIPE)])


@functools.partial(
    pl.kernel,
    out_type=jax.ShapeDtypeStruct((NP, DIM), jnp.float32),
    mesh=_mesh,
    compiler_params=_cp,
    scratch_types=[
        pltpu.VMEM((CH,), jnp.int32),
        pltpu.VMEM((CH,), jnp.int32),
        pltpu.VMEM((1, CH), jnp.int32),
        pltpu.VMEM((CH, DIM), jnp.float32),
        pltpu.VMEM_SHARED((R_ACC, DIM), jnp.float32),
    ],
)
def _prop_kernel(p_hbm, e_hbm, y_hbm, tbuf, hbuf, xbuf, rows, acc):
    c = lax.axis_index("c")
    s = lax.axis_index("s")

    @pl.loop(0, CH)
    def _(r):
        for q in range(DIM // 16):
            rows[r, pl.ds(q * 16, 16)] = jnp.zeros((16,), jnp.float32)

    @pl.loop(0, STRIPE // CH)
    def _(k):
        pltpu.sync_copy(rows, acc.at[pl.ds(s * STRIPE + k * CH, CH)])

    plsc.subcore_barrier()

    lane = lax.iota(jnp.int32, 16)
    lo = c * HALF

    @pl.loop(0, NCH)
    def _(i):
        base = s * (NCH * CH) + i * CH
        pltpu.sync_copy(e_hbm.at[1, pl.ds(base, CH)], tbuf)
        pltpu.sync_copy(e_hbm.at[0, pl.ds(base, CH)], hbuf)
        pltpu.sync_copy(p_hbm.at[tbuf], rows)
        for j in range(CH // 16):
            v = hbuf[pl.ds(j * 16, 16)] - lo
            ok = (v >= 0) & (v < HALF)
            trash = HALF + lane + j * 16
            xbuf[0, pl.ds(j * 16, 16)] = jnp.where(ok, v, trash)
        pltpu.sync_copy(rows, acc.at[xbuf.at[0]], add=True)

    plsc.subcore_barrier()

    gbase = c * HALF + s * STRIPE

    @pl.when(s < 15)
    def _():
        pltpu.sync_copy(acc.at[pl.ds(s * STRIPE, STRIPE)],
                        y_hbm.at[pl.ds(gbase, STRIPE)])

    @pl.when(s == 15)
    def _():
        pltpu.sync_copy(acc.at[pl.ds(s * STRIPE, TAIL)],
                        y_hbm.at[pl.ds(gbase, TAIL)])


def _s0_body(x_ref, dh_ref, dt_ref, p_ref, dhi_ref, dti_ref):
    dhi = lax.rsqrt(jnp.maximum(dh_ref[...], 1.0))
    dti = lax.rsqrt(jnp.maximum(dt_ref[...], 1.0))
    p_ref[...] = x_ref[...] * dti
    dhi_ref[...] = dhi
    dti_ref[...] = dti


def _s1_body(y_ref, x_ref, dhi_ref, dti_ref, p_ref, a_ref):
    x1 = dhi_ref[...] * y_ref[...]
    p_ref[...] = dti_ref[...] * x1
    a_ref[...] = 2.0 * x_ref[...] + 2.0 * x1


def _s2_body(a_ref, y_ref, dhi_ref, o_ref):
    o_ref[...] = (a_ref[...] + dhi_ref[...] * y_ref[...]) * (1.0 / 3.0)


_mat = pl.BlockSpec((RB, DIM), lambda i: (i, 0))
_col = pl.BlockSpec((RB, 1), lambda i: (i, 0))
_fmat = jax.ShapeDtypeStruct((NP, DIM), jnp.float32)
_fcol = jax.ShapeDtypeStruct((NP, 1), jnp.float32)

_s0 = pl.pallas_call(
    _s0_body, grid=(NP // RB,),
    in_specs=[_mat, _col, _col],
    out_specs=[_mat, _col, _col],
    out_shape=[_fmat, _fcol, _fcol],
)

_s1 = pl.pallas_call(
    _s1_body, grid=(NP // RB,),
    in_specs=[_mat, _mat, _col, _col],
    out_specs=[_mat, _mat],
    out_shape=[_fmat, _fmat],
)

_s2 = pl.pallas_call(
    _s2_body, grid=(NP // RB,),
    in_specs=[_mat, _mat, _col],
    out_specs=_mat,
    out_shape=_fmat,
)


def kernel(u_emb, i_emb, edge_index):
    e = edge_index.astype(jnp.int32)
    e = jnp.concatenate(
        [e, jnp.full((2, EP - N_EDGES), N_NODES, jnp.int32)], axis=1)
    x = jnp.concatenate([u_emb, i_emb], axis=0)
    x = jnp.pad(x, ((0, NP - N_NODES), (0, 0)))

    deg = _degree_kernel(e)
    degh = jnp.pad(deg[0, :N_NODES], (0, NP - N_NODES)).reshape(NP, 1)
    degt = jnp.pad(deg[1, :N_NODES], (0, NP - N_NODES)).reshape(NP, 1)

    p0, dhi, dti = _s0(x, degh, degt)
    y1 = _prop_kernel(p0, e)
    p1, acc1 = _s1(y1, x, dhi, dti)
    y2 = _prop_kernel(p1, e)
    out = _s2(acc1, y2, dhi)
    return out[:N_NODES]
